# Initial kernel scaffold; baseline (speedup 1.0000x reference)
import jax
import jax.numpy as jnp
from jax.experimental import pallas as pl


def kernel(classifications, regressions, anchors, annotations):
    raise NotImplementedError("write your pallas kernel here")


# speedup vs baseline: 4.0933x; 4.0933x over previous
import functools

import jax
import jax.numpy as jnp
from jax.experimental import pallas as pl
from jax.experimental.pallas import tpu as pltpu

_NEG_CODE = 128.0
_IGN_CODE = 1024.0


def _assign_body(nvalid, naboxes, sbox_ref, axs_ref, regt_ref, code_ref, stats_ref):
    ax1 = axs_ref[0]
    ay1 = axs_ref[1]
    ax2 = axs_ref[2]
    ay2 = axs_ref[3]
    aw = ax2 - ax1
    ah = ay2 - ay1
    aarea = aw * ah

    def sc(b, r):
        return sbox_ref[0, 0, b * 6 + r]

    def inter_union(b):
        bx1, by1, bx2, by2 = sc(b, 0), sc(b, 1), sc(b, 2), sc(b, 3)
        iw = jnp.maximum(jnp.minimum(ax2, bx2) - jnp.maximum(ax1, bx1), 0.0)
        ih = jnp.maximum(jnp.minimum(ay2, by2) - jnp.maximum(ay1, by1), 0.0)
        inter = iw * ih
        ua = jnp.maximum(aarea + (sc(b, 5) - inter), 1e-8)
        return inter, ua

    best_i, best_u = inter_union(0)
    shp = best_i.shape
    sel_x1 = jnp.full(shp, sc(0, 0), jnp.float32)
    sel_y1 = jnp.full(shp, sc(0, 1), jnp.float32)
    sel_x2 = jnp.full(shp, sc(0, 2), jnp.float32)
    sel_y2 = jnp.full(shp, sc(0, 3), jnp.float32)
    sel_cl = jnp.full(shp, sc(0, 4), jnp.float32)
    for b in range(1, naboxes):
        inter, ua = inter_union(b)
        gt = inter * best_u > best_i * ua
        best_i = jnp.where(gt, inter, best_i)
        best_u = jnp.where(gt, ua, best_u)
        sel_x1 = jnp.where(gt, sc(b, 0), sel_x1)
        sel_y1 = jnp.where(gt, sc(b, 1), sel_y1)
        sel_x2 = jnp.where(gt, sc(b, 2), sel_x2)
        sel_y2 = jnp.where(gt, sc(b, 3), sel_y2)
        sel_cl = jnp.where(gt, sc(b, 4), sel_cl)

    iou_max = best_i / best_u
    rows = jax.lax.broadcasted_iota(jnp.int32, shp, 0)
    lanes = jax.lax.broadcasted_iota(jnp.int32, shp, 1)
    inb = rows * 128 + lanes < nvalid
    pos = (iou_max >= 0.5) & inb
    neg = (iou_max < 0.4) & inb

    code = jnp.where(pos, sel_cl, jnp.where(neg, _NEG_CODE, _IGN_CODE))
    code_ref[0] = code

    gw = sel_x2 - sel_x1
    gh = sel_y2 - sel_y1
    gcx = sel_x1 + 0.5 * gw
    gcy = sel_y1 + 0.5 * gh
    inv_aw = 1.0 / aw
    inv_ah = 1.0 / ah
    acx = ax1 + 0.5 * aw
    acy = ay1 + 0.5 * ah
    t0 = (gcx - acx) * inv_aw * 10.0
    t1 = (gcy - acy) * inv_ah * 10.0
    t2 = jnp.log(jnp.maximum(gw, 1.0) * inv_aw) * 5.0
    t3 = jnp.log(jnp.maximum(gh, 1.0) * inv_ah) * 5.0
    rloss = jnp.zeros(shp, jnp.float32)
    for k, t in enumerate((t0, t1, t2, t3)):
        d = jnp.abs(t - regt_ref[0, k])
        rloss = rloss + jnp.where(d <= 1.0 / 9.0, 4.5 * d * d, d - 0.5 / 9.0)
    posf = pos.astype(jnp.float32)
    rloss = jnp.where(pos, rloss, 0.0)
    stats_ref[0, 0:1, :] = jnp.sum(posf, axis=0, keepdims=True)
    stats_ref[0, 1:2, :] = jnp.sum(rloss, axis=0, keepdims=True)


def _focal_body(cls_ref, code_ref, out_ref):
    i = pl.program_id(1)
    x = cls_ref[0]
    c = jnp.clip(x, 1e-4, 1.0 - 1e-4)
    code = code_ref[0]
    lane = jax.lax.broadcasted_iota(jnp.int32, x.shape, 1).astype(jnp.float32)
    sel1 = lane == code
    valid = code < 200.0
    li = jnp.where(sel1, c, 1.0 - c)
    fw = 1.0 - li
    af = jnp.where(sel1, -0.75, -0.25)
    loss = af * fw * fw * jnp.log(li)
    loss = jnp.where(valid, loss, 0.0)
    partial = jnp.sum(loss, axis=0, keepdims=True)[None]

    @pl.when(i == 0)
    def _():
        out_ref[...] = partial

    @pl.when(i != 0)
    def _():
        out_ref[...] += partial


@jax.jit
def kernel(classifications, regressions, anchors, annotations):
    B, N, C = classifications.shape
    A = annotations.shape[1]
    npad = ((N + 1023) // 1024) * 1024
    R = npad // 128

    anc = jnp.pad(anchors[0], ((0, npad - N), (0, 0)))
    axs = anc.T.reshape(4, R, 128)

    ann = annotations
    fx1, fy1, fx2, fy2 = ann[:, :, 4], ann[:, :, 5], ann[:, :, 6], ann[:, :, 7]
    barea = (fx2 - fx1) * (fy2 - fy1)
    sbox = jnp.stack([fx1, fy1, fx2, fy2, ann[:, :, 8], barea], axis=2)
    sbox = sbox.reshape(B, 1, A * 6)

    regt = regressions[:, :, 0:4].transpose(0, 2, 1)
    regt = jnp.pad(regt, ((0, 0), (0, 0), (0, npad - N))).reshape(B, 4, R, 128)

    code, stats = pl.pallas_call(
        functools.partial(_assign_body, N, A),
        grid=(B,),
        in_specs=[
            pl.BlockSpec((1, 1, A * 6), lambda j: (j, 0, 0), memory_space=pltpu.SMEM),
            pl.BlockSpec((4, R, 128), lambda j: (0, 0, 0)),
            pl.BlockSpec((1, 4, R, 128), lambda j: (j, 0, 0, 0)),
        ],
        out_specs=[
            pl.BlockSpec((1, R, 128), lambda j: (j, 0, 0)),
            pl.BlockSpec((1, 2, 128), lambda j: (j, 0, 0)),
        ],
        out_shape=[
            jax.ShapeDtypeStruct((B, R, 128), jnp.float32),
            jax.ShapeDtypeStruct((B, 2, 128), jnp.float32),
        ],
    )(sbox, axs, regt)

    nb = 2048
    nblocks = (N + nb - 1) // nb
    code3 = code.reshape(B, npad, 1)
    csum = pl.pallas_call(
        _focal_body,
        grid=(B, nblocks),
        in_specs=[
            pl.BlockSpec((1, nb, C), lambda j, i: (j, i, 0)),
            pl.BlockSpec((1, nb, 1), lambda j, i: (j, i, 0)),
        ],
        out_specs=pl.BlockSpec((1, 1, C), lambda j, i: (j, 0, 0)),
        out_shape=jax.ShapeDtypeStruct((B, 1, C), jnp.float32),
    )(classifications, code3)

    cls_sum = jnp.sum(csum, axis=(1, 2))
    npos = jnp.sum(stats[:, 0, :], axis=-1)
    regsum = jnp.sum(stats[:, 1, :], axis=-1)
    cls_loss = cls_sum / jnp.clip(npos, 1.0, None)
    reg_loss = jnp.where(npos > 0, regsum / jnp.maximum(npos * 4.0, 1.0), 0.0)
    return (cls_loss.mean(keepdims=True), reg_loss.mean(keepdims=True))
